# baseline (speedup 1.0000x reference)
import functools

import jax
import jax.numpy as jnp
from jax import lax
from jax.experimental import pallas as pl
from jax.experimental.pallas import tpu as pltpu
from jax.experimental.pallas import tpu_sc as plsc

_RANK = 8
_DIM = 64
_LANES = 16
_CHUNK = 128
_APAD = 1000064


def _lora_embed_body(steps, x_hbm, w_hbm, aflat_hbm, bs_hbm, out_hbm,
                     idx_v, xe0, xe1, af0, af1, wr0, wr1, ob0, ob1,
                     bs_v, sg0, sg1, so0, so1):
  num_cores = 2
  cid = lax.axis_index("c")
  sid = lax.axis_index("s")
  wid = sid * num_cores + cid

  xe = [xe0, xe1]
  af = [af0, af1]
  wr = [wr0, wr1]
  ob = [ob0, ob1]
  sg = [sg0, sg1]
  so = [so0, so1]

  pltpu.sync_copy(x_hbm.at[pl.ds(wid * steps, steps)], idx_v)
  pltpu.sync_copy(bs_hbm, bs_v)

  bsv = [[plsc.pack(bs_v[pl.ds(r * _DIM + (2 * h) * _LANES, _LANES)],
                    bs_v[pl.ds(r * _DIM + (2 * h + 1) * _LANES, _LANES)],
                    format=plsc.PackFormat.INTERLEAVED)
          for h in range(2)]
         for r in range(_RANK)]
  cidx = [jnp.full((_LANES,), l, jnp.int32) for l in range(_LANES)]
  roff = [jnp.full((_LANES,), r * _APAD, jnp.int32) for r in range(_RANK)]

  def build_and_fire(t, b):
    for m in range(8):
      xv = idx_v[t, pl.ds(m * _LANES, _LANES)]
      for r in range(_RANK):
        xe[b][r, pl.ds(m * _LANES, _LANES)] = xv + roff[r]
    pltpu.async_copy(w_hbm.at[idx_v.at[t]], wr[b], sg[b])
    for r in range(_RANK):
      pltpu.async_copy(aflat_hbm.at[xe[b].at[r]], af[b].at[r], sg[b])

  def wait_gathers(t, b):
    pltpu.make_async_copy(w_hbm.at[idx_v.at[t]], wr[b], sg[b]).wait()
    for r in range(_RANK):
      pltpu.make_async_copy(aflat_hbm.at[xe[b].at[r]], af[b].at[r],
                            sg[b]).wait()

  def out_slice(t):
    return out_hbm.at[pl.ds((wid * steps + t) * _CHUNK, _CHUNK)]

  def compute(t, b):
    def krow(k, c):
      av = [af[b][r, pl.ds(k * _LANES, _LANES)] for r in range(_RANK)]
      for i in range(_LANES):
        row = 16 * k + i
        asp = [plsc.pack(s, s, format=plsc.PackFormat.INTERLEAVED)
               for s in (av[r].at[cidx[i]].get(mode="promise_in_bounds")
                         for r in range(_RANK))]
        for h in range(2):
          acc = asp[0] * bsv[0][h]
          for r in range(1, _RANK):
            acc = acc + asp[r] * bsv[r][h]
          e0, e1 = plsc.unpack(acc, format=plsc.PackFormat.INTERLEAVED)
          g0 = 2 * h
          g1 = 2 * h + 1
          ob[b][row, pl.ds(g0 * _LANES, _LANES)] = (
              wr[b][row, pl.ds(g0 * _LANES, _LANES)] + e0)
          ob[b][row, pl.ds(g1 * _LANES, _LANES)] = (
              wr[b][row, pl.ds(g1 * _LANES, _LANES)] + e1)
      return c

    lax.fori_loop(0, _CHUNK // 16, krow, 0)

  build_and_fire(0, 0)

  def body(u, c):
    t0 = 2 * u
    t1 = 2 * u + 1
    build_and_fire(t1, 1)
    wait_gathers(t0, 0)

    @pl.when(u > 0)
    def _():
      pltpu.make_async_copy(ob[0], out_slice(t0 - 2), so[0]).wait()

    compute(t0, 0)
    pltpu.async_copy(ob[0], out_slice(t0), so[0])

    @pl.when(t0 + 2 < steps)
    def _():
      build_and_fire(t0 + 2, 0)

    wait_gathers(t1, 1)

    @pl.when(u > 0)
    def _():
      pltpu.make_async_copy(ob[1], out_slice(t1 - 2), so[1]).wait()

    compute(t1, 1)
    pltpu.async_copy(ob[1], out_slice(t1), so[1])
    return c

  lax.fori_loop(0, steps // 2, body, 0)
  pltpu.make_async_copy(ob[0], out_slice(steps - 2), so[0]).wait()
  pltpu.make_async_copy(ob[1], out_slice(steps - 1), so[1]).wait()


def kernel(x, weight, lora_A, lora_B):
  scaling = _RANK / (_RANK ** 0.5)
  n = x.shape[0] * x.shape[1]
  num_workers = 32
  rows_per_w = n // num_workers
  steps = rows_per_w // _CHUNK
  nchunks = num_workers * steps

  xf = x.reshape(n).astype(jnp.int32).reshape(nchunks, _CHUNK)
  bs = (lora_B * scaling).astype(jnp.float32).reshape(_RANK * _DIM)
  a_pad = jnp.pad(lora_A.T, ((0, 0), (0, _APAD - lora_A.shape[0])))
  a_flat = a_pad.reshape(_RANK * _APAD)

  mesh = plsc.VectorSubcoreMesh(core_axis_name="c", subcore_axis_name="s")
  run = pl.kernel(
      functools.partial(_lora_embed_body, steps),
      out_type=jax.ShapeDtypeStruct((n, _DIM), jnp.float32),
      mesh=mesh,
      compiler_params=pltpu.CompilerParams(use_tc_tiling_on_sc=False,
                                           needs_layout_passes=False),
      scratch_types=[
          pltpu.VMEM((steps, _CHUNK), jnp.int32),
          pltpu.VMEM((_RANK, _CHUNK), jnp.int32),
          pltpu.VMEM((_RANK, _CHUNK), jnp.int32),
          pltpu.VMEM((_RANK, _CHUNK), jnp.float32),
          pltpu.VMEM((_RANK, _CHUNK), jnp.float32),
          pltpu.VMEM((_CHUNK, _DIM), jnp.float32),
          pltpu.VMEM((_CHUNK, _DIM), jnp.float32),
          pltpu.VMEM((_CHUNK, _DIM), jnp.float32),
          pltpu.VMEM((_CHUNK, _DIM), jnp.float32),
          pltpu.VMEM((_RANK * _DIM,), jnp.float32),
          pltpu.SemaphoreType.DMA,
          pltpu.SemaphoreType.DMA,
          pltpu.SemaphoreType.DMA,
          pltpu.SemaphoreType.DMA,
      ],
  )
  out = run(xf, weight, a_flat, bs)
  return out.reshape(x.shape[0], x.shape[1], _DIM)


# speedup vs baseline: 1.0753x; 1.0753x over previous
import functools

import jax
import jax.numpy as jnp
from jax import lax
from jax.experimental import pallas as pl
from jax.experimental.pallas import tpu as pltpu
from jax.experimental.pallas import tpu_sc as plsc

_RANK = 8
_DIM = 64
_LANES = 16
_CHUNK = 128
_APAD = 1000000


def _lora_embed_body(steps, x_hbm, w_hbm, aflat_hbm, bs_hbm, out_hbm,
                     idx_v, xs0, xs1, xe0, xe1, af0, af1, wr0, wr1, ob0, ob1,
                     bs_v, sg0, sg1, so0, so1):
  num_cores = 2
  cid = lax.axis_index("c")
  sid = lax.axis_index("s")
  wid = sid * num_cores + cid

  xs = [xs0, xs1]
  xe = [xe0, xe1]
  af = [af0, af1]
  wr = [wr0, wr1]
  ob = [ob0, ob1]
  sg = [sg0, sg1]
  so = [so0, so1]

  pltpu.sync_copy(x_hbm.at[pl.ds(wid * steps, steps)], idx_v)
  pltpu.sync_copy(bs_hbm, bs_v)

  bsv = [[plsc.pack(bs_v[pl.ds(r * _DIM + (2 * h) * _LANES, _LANES)],
                    bs_v[pl.ds(r * _DIM + (2 * h + 1) * _LANES, _LANES)],
                    format=plsc.PackFormat.INTERLEAVED)
          for h in range(2)]
         for r in range(_RANK)]
  cidx = [jnp.full((_LANES,), l, jnp.int32) for l in range(_LANES)]
  ones = jnp.full((_LANES,), 1, jnp.int32)
  roff = [jnp.full((_LANES,), r * _APAD, jnp.int32) for r in range(_RANK)]

  def build_and_fire(t, b):
    for m in range(8):
      xv = idx_v[t, pl.ds(m * _LANES, _LANES)]
      xs[b][pl.ds(m * _LANES, _LANES)] = lax.shift_right_logical(xv, ones)
      for r in range(_RANK):
        xe[b][r, pl.ds(m * _LANES, _LANES)] = xv + roff[r]
    pltpu.async_copy(w_hbm.at[xs[b]], wr[b], sg[b])
    for r in range(_RANK):
      pltpu.async_copy(aflat_hbm.at[xe[b].at[r]], af[b].at[r], sg[b])

  def wait_gathers(t, b):
    pltpu.make_async_copy(w_hbm.at[xs[b]], wr[b], sg[b]).wait()
    for r in range(_RANK):
      pltpu.make_async_copy(aflat_hbm.at[xe[b].at[r]], af[b].at[r],
                            sg[b]).wait()

  def out_slice(t):
    return out_hbm.at[pl.ds((wid * steps + t) * _CHUNK, _CHUNK)]

  def compute(t, b):
    def krow(k, c):
      av = [af[b][r, pl.ds(k * _LANES, _LANES)] for r in range(_RANK)]
      xvk = idx_v[t, pl.ds(k * _LANES, _LANES)]
      for i in range(_LANES):
        row = 16 * k + i
        off = lax.mul(lax.bitwise_and(xvk[i], 1), _DIM)
        asp = [plsc.pack(s, s, format=plsc.PackFormat.INTERLEAVED)
               for s in (av[r].at[cidx[i]].get(mode="promise_in_bounds")
                         for r in range(_RANK))]
        for h in range(2):
          acc = asp[0] * bsv[0][h]
          for r in range(1, _RANK):
            acc = acc + asp[r] * bsv[r][h]
          e0, e1 = plsc.unpack(acc, format=plsc.PackFormat.INTERLEAVED)
          g0 = 2 * h
          g1 = 2 * h + 1
          ob[b][row, pl.ds(g0 * _LANES, _LANES)] = (
              wr[b][row, pl.ds(off + g0 * _LANES, _LANES)] + e0)
          ob[b][row, pl.ds(g1 * _LANES, _LANES)] = (
              wr[b][row, pl.ds(off + g1 * _LANES, _LANES)] + e1)
      return c

    lax.fori_loop(0, _CHUNK // 16, krow, 0)

  build_and_fire(0, 0)

  def body(u, c):
    t0 = 2 * u
    t1 = 2 * u + 1
    build_and_fire(t1, 1)
    wait_gathers(t0, 0)

    @pl.when(u > 0)
    def _():
      pltpu.make_async_copy(ob[0], out_slice(t0 - 2), so[0]).wait()

    compute(t0, 0)
    pltpu.async_copy(ob[0], out_slice(t0), so[0])

    @pl.when(t0 + 2 < steps)
    def _():
      build_and_fire(t0 + 2, 0)

    wait_gathers(t1, 1)

    @pl.when(u > 0)
    def _():
      pltpu.make_async_copy(ob[1], out_slice(t1 - 2), so[1]).wait()

    compute(t1, 1)
    pltpu.async_copy(ob[1], out_slice(t1), so[1])
    return c

  lax.fori_loop(0, steps // 2, body, 0)
  pltpu.make_async_copy(ob[0], out_slice(steps - 2), so[0]).wait()
  pltpu.make_async_copy(ob[1], out_slice(steps - 1), so[1]).wait()


def kernel(x, weight, lora_A, lora_B):
  scaling = _RANK / (_RANK ** 0.5)
  n = x.shape[0] * x.shape[1]
  num_workers = 32
  rows_per_w = n // num_workers
  steps = rows_per_w // _CHUNK
  nchunks = num_workers * steps

  xf = x.reshape(n).astype(jnp.int32).reshape(nchunks, _CHUNK)
  bs = (lora_B * scaling).astype(jnp.float32).reshape(_RANK * _DIM)
  a_flat = jnp.concatenate([lora_A[:, r] for r in range(_RANK)])
  w_pairs = weight.reshape(weight.shape[0] // 2, 2 * _DIM)

  mesh = plsc.VectorSubcoreMesh(core_axis_name="c", subcore_axis_name="s")
  run = pl.kernel(
      functools.partial(_lora_embed_body, steps),
      out_type=jax.ShapeDtypeStruct((n, _DIM), jnp.float32),
      mesh=mesh,
      compiler_params=pltpu.CompilerParams(use_tc_tiling_on_sc=False,
                                           needs_layout_passes=False),
      scratch_types=[
          pltpu.VMEM((steps, _CHUNK), jnp.int32),
          pltpu.VMEM((_CHUNK,), jnp.int32),
          pltpu.VMEM((_CHUNK,), jnp.int32),
          pltpu.VMEM((_RANK, _CHUNK), jnp.int32),
          pltpu.VMEM((_RANK, _CHUNK), jnp.int32),
          pltpu.VMEM((_RANK, _CHUNK), jnp.float32),
          pltpu.VMEM((_RANK, _CHUNK), jnp.float32),
          pltpu.VMEM((_CHUNK, 2 * _DIM), jnp.float32),
          pltpu.VMEM((_CHUNK, 2 * _DIM), jnp.float32),
          pltpu.VMEM((_CHUNK, _DIM), jnp.float32),
          pltpu.VMEM((_CHUNK, _DIM), jnp.float32),
          pltpu.VMEM((_RANK * _DIM,), jnp.float32),
          pltpu.SemaphoreType.DMA,
          pltpu.SemaphoreType.DMA,
          pltpu.SemaphoreType.DMA,
          pltpu.SemaphoreType.DMA,
      ],
  )
  out = run(xf, w_pairs, a_flat, bs)
  return out.reshape(x.shape[0], x.shape[1], _DIM)


# speedup vs baseline: 1.1000x; 1.0230x over previous
import functools

import jax
import jax.numpy as jnp
from jax import lax
from jax.experimental import pallas as pl
from jax.experimental.pallas import tpu as pltpu
from jax.experimental.pallas import tpu_sc as plsc

_RANK = 8
_DIM = 64
_LANES = 16
_CHUNK = 128


def _lora_embed_body(steps, x_hbm, w_hbm, a_hbm, bs_hbm, out_hbm,
                     idx_v, xe0, xe1, af0, af1, wr0, wr1, ob0, ob1,
                     bs_v, sg0, sg1, so0, so1):
  num_cores = 2
  cid = lax.axis_index("c")
  sid = lax.axis_index("s")
  wid = sid * num_cores + cid

  xe = [xe0, xe1]
  af = [af0, af1]
  wr = [wr0, wr1]
  ob = [ob0, ob1]
  sg = [sg0, sg1]
  so = [so0, so1]

  pltpu.sync_copy(x_hbm.at[pl.ds(wid * steps, steps)], idx_v)
  pltpu.sync_copy(bs_hbm, bs_v)

  bsv = [[plsc.pack(bs_v[pl.ds(r * _DIM + (2 * h) * _LANES, _LANES)],
                    bs_v[pl.ds(r * _DIM + (2 * h + 1) * _LANES, _LANES)],
                    format=plsc.PackFormat.INTERLEAVED)
          for h in range(2)]
         for r in range(_RANK)]
  cidx = [jnp.full((_LANES,), l, jnp.int32) for l in range(_LANES)]
  roff = [jnp.full((_LANES,), r, jnp.int32) for r in range(_RANK)]
  three = jnp.full((_LANES,), 3, jnp.int32)

  def build_and_fire(t, b):
    for m in range(8):
      xb = lax.shift_left(idx_v[t, pl.ds(m * _LANES, _LANES)], three)
      for r in range(_RANK):
        xe[b][r, pl.ds(m * _LANES, _LANES)] = xb + roff[r]
    pltpu.async_copy(w_hbm.at[idx_v.at[t]], wr[b], sg[b])
    for r in range(_RANK):
      pltpu.async_copy(a_hbm.at[xe[b].at[r]], af[b].at[r], sg[b])

  def wait_gathers(t, b):
    pltpu.make_async_copy(w_hbm.at[idx_v.at[t]], wr[b], sg[b]).wait()
    for r in range(_RANK):
      pltpu.make_async_copy(a_hbm.at[xe[b].at[r]], af[b].at[r],
                            sg[b]).wait()

  def out_slice(t):
    return out_hbm.at[pl.ds((wid * steps + t) * _CHUNK, _CHUNK)]

  def compute(t, b):
    def krow(k, c):
      av = [af[b][r, pl.ds(k * _LANES, _LANES)] for r in range(_RANK)]
      for i in range(_LANES):
        row = 16 * k + i
        asp = [plsc.pack(s, s, format=plsc.PackFormat.INTERLEAVED)
               for s in (av[r].at[cidx[i]].get(mode="promise_in_bounds")
                         for r in range(_RANK))]
        for h in range(2):
          acc = asp[0] * bsv[0][h]
          for r in range(1, _RANK):
            acc = acc + asp[r] * bsv[r][h]
          e0, e1 = plsc.unpack(acc, format=plsc.PackFormat.INTERLEAVED)
          g0 = 2 * h
          g1 = 2 * h + 1
          ob[b][row, pl.ds(g0 * _LANES, _LANES)] = (
              wr[b][row, pl.ds(g0 * _LANES, _LANES)] + e0)
          ob[b][row, pl.ds(g1 * _LANES, _LANES)] = (
              wr[b][row, pl.ds(g1 * _LANES, _LANES)] + e1)
      return c

    lax.fori_loop(0, _CHUNK // 16, krow, 0)

  build_and_fire(0, 0)

  def body(u, c):
    t0 = 2 * u
    t1 = 2 * u + 1
    build_and_fire(t1, 1)
    wait_gathers(t0, 0)

    @pl.when(u > 0)
    def _():
      pltpu.make_async_copy(ob[0], out_slice(t0 - 2), so[0]).wait()

    compute(t0, 0)
    pltpu.async_copy(ob[0], out_slice(t0), so[0])

    @pl.when(t0 + 2 < steps)
    def _():
      build_and_fire(t0 + 2, 0)

    wait_gathers(t1, 1)

    @pl.when(u > 0)
    def _():
      pltpu.make_async_copy(ob[1], out_slice(t1 - 2), so[1]).wait()

    compute(t1, 1)
    pltpu.async_copy(ob[1], out_slice(t1), so[1])
    return c

  lax.fori_loop(0, steps // 2, body, 0)
  pltpu.make_async_copy(ob[0], out_slice(steps - 2), so[0]).wait()
  pltpu.make_async_copy(ob[1], out_slice(steps - 1), so[1]).wait()


def kernel(x, weight, lora_A, lora_B):
  scaling = _RANK / (_RANK ** 0.5)
  n = x.shape[0] * x.shape[1]
  num_workers = 32
  rows_per_w = n // num_workers
  steps = rows_per_w // _CHUNK
  nchunks = num_workers * steps

  xf = x.reshape(n).astype(jnp.int32).reshape(nchunks, _CHUNK)
  bs = (lora_B * scaling).astype(jnp.float32).reshape(_RANK * _DIM)
  a_flat = lora_A.reshape(lora_A.shape[0] * _RANK)
  mesh = plsc.VectorSubcoreMesh(core_axis_name="c", subcore_axis_name="s")
  run = pl.kernel(
      functools.partial(_lora_embed_body, steps),
      out_type=jax.ShapeDtypeStruct((n, _DIM), jnp.float32),
      mesh=mesh,
      compiler_params=pltpu.CompilerParams(use_tc_tiling_on_sc=False,
                                           needs_layout_passes=False),
      scratch_types=[
          pltpu.VMEM((steps, _CHUNK), jnp.int32),
          pltpu.VMEM((_RANK, _CHUNK), jnp.int32),
          pltpu.VMEM((_RANK, _CHUNK), jnp.int32),
          pltpu.VMEM((_RANK, _CHUNK), jnp.float32),
          pltpu.VMEM((_RANK, _CHUNK), jnp.float32),
          pltpu.VMEM((_CHUNK, _DIM), jnp.float32),
          pltpu.VMEM((_CHUNK, _DIM), jnp.float32),
          pltpu.VMEM((_CHUNK, _DIM), jnp.float32),
          pltpu.VMEM((_CHUNK, _DIM), jnp.float32),
          pltpu.VMEM((_RANK * _DIM,), jnp.float32),
          pltpu.SemaphoreType.DMA,
          pltpu.SemaphoreType.DMA,
          pltpu.SemaphoreType.DMA,
          pltpu.SemaphoreType.DMA,
      ],
  )
  out = run(xf, weight, a_flat, bs)
  return out.reshape(x.shape[0], x.shape[1], _DIM)
